# Initial kernel scaffold; baseline (speedup 1.0000x reference)
import jax
import jax.numpy as jnp
from jax.experimental import pallas as pl


def kernel(x_indices, x_scale, table):
    raise NotImplementedError("write your pallas kernel here")


# speedup vs baseline: 2.7587x; 2.7587x over previous
import functools

import jax
import jax.numpy as jnp
from jax import lax
from jax.experimental import pallas as pl
from jax.experimental.pallas import tpu as pltpu
from jax.experimental.pallas import tpu_sc as plsc

VOCAB = 64
D = 128
BATCH = 4096
HIST = 50
TOTAL = BATCH * HIST
NC = 2
NS = 16
NW = NC * NS
PER_W = TOTAL // NW
CH = 256
NCH = PER_W // CH
L = 16


@functools.partial(
    pl.kernel,
    out_type=jax.ShapeDtypeStruct((TOTAL, D), jnp.float32),
    mesh=plsc.VectorSubcoreMesh(core_axis_name="c", subcore_axis_name="s"),
    scratch_types=[
        pltpu.VMEM_SHARED((VOCAB, D), jnp.float32),
        pltpu.VMEM((CH,), jnp.int32),
        pltpu.VMEM((CH,), jnp.float32),
        pltpu.VMEM((CH, D), jnp.float32),
        pltpu.SemaphoreType.DMA,
    ],
)
def _lookup_scale(idx_hbm, scale_hbm, table_hbm, out_hbm,
                  table_sh, idx_v, scale_v, rows_v, sem):
    sid = lax.axis_index("s")
    wid = sid * NC + lax.axis_index("c")
    base = pl.multiple_of(wid * PER_W, CH)

    @pl.when(sid == 0)
    def _():
        pltpu.sync_copy(table_hbm, table_sh)

    plsc.subcore_barrier()

    def chunk_body(c, _):
        off = pl.multiple_of(base + c * CH, CH)
        pltpu.sync_copy(idx_hbm.at[pl.ds(off, CH)], idx_v)
        pltpu.sync_copy(scale_hbm.at[pl.ds(off, CH)], scale_v)
        pltpu.async_copy(table_sh.at[idx_v], rows_v, sem).wait()

        dnums = lax.GatherDimensionNumbers(
            offset_dims=(), collapsed_slice_dims=(0,), start_index_map=(0,))

        def group_body(g, _):
            r0 = g * L
            sv = scale_v[pl.ds(r0, L)]
            for j in range(L):
                i = r0 + j
                sj = lax.gather(
                    sv, jnp.full((L, 1), j, jnp.int32), dnums, (1,),
                    mode=lax.GatherScatterMode.PROMISE_IN_BOUNDS)
                for d8 in range(D // L):
                    sl = pl.ds(d8 * L, L)
                    rows_v[i, sl] = rows_v[i, sl] * sj
            return 0

        lax.fori_loop(0, CH // L, group_body, 0)
        pltpu.sync_copy(rows_v, out_hbm.at[pl.ds(off, CH)])
        return 0

    lax.fori_loop(0, NCH, chunk_body, 0)


def kernel(x_indices, x_scale, table):
    idx = x_indices.reshape(TOTAL).astype(jnp.int32)
    scale = x_scale.reshape(TOTAL)
    out = _lookup_scale(idx, scale, table)
    return out.reshape(BATCH, HIST, D)
